# Initial kernel scaffold; baseline (speedup 1.0000x reference)
import functools

import jax
import jax.numpy as jnp
from jax import lax
from jax.experimental import pallas as pl
from jax.experimental.pallas import tpu as pltpu
from jax.experimental.pallas import tpu_sc as plsc

NC = 2
NS = 16
L = 16
NW = NC * NS

N_ATOMS = 100000
N_BONDS = 1600000
BT = N_BONDS // NW
CHUNK = 2000
N_CHUNKS = BT // CHUNK
GROUPS = CHUNK // L

_i32 = jnp.int32
_f32 = jnp.float32


def _sc_body(coords_hbm, bidx_hbm, b0_hbm, k_hbm, out_hbm,
             coords_sh, partials_sh,
             bidx_v, b0_v, k_v, rows_v, red_v, acc_v, tot_v, sem0):
    cid = lax.axis_index("c")
    sid = lax.axis_index("s")
    wid = cid * NS + sid

    @pl.when(sid == 0)
    def _():
        pltpu.sync_copy(coords_hbm, coords_sh)

    plsc.subcore_barrier()

    col0 = jnp.zeros((L,), _i32)
    col1 = jnp.ones((L,), _i32)
    col2 = jnp.full((L,), 2, _i32)
    lanes = lax.iota(_i32, L)

    def chunk_step(t, acc):
        base = wid * BT + t * CHUNK
        pltpu.sync_copy(bidx_hbm.at[pl.ds(base * 2, CHUNK * 2)], bidx_v)
        pltpu.sync_copy(b0_hbm.at[pl.ds(base, CHUNK)], b0_v)
        pltpu.sync_copy(k_hbm.at[pl.ds(base, CHUNK)], k_v)
        pltpu.async_copy(coords_sh.at[bidx_v], rows_v, sem0).wait()

        def group_step(g, acc_in):
            rows = g * (2 * L) + lanes * 2
            x0 = plsc.load_gather(rows_v, [rows, col0])
            y0 = plsc.load_gather(rows_v, [rows, col1])
            z0 = plsc.load_gather(rows_v, [rows, col2])
            x1 = plsc.load_gather(rows_v, [rows + 1, col0])
            y1 = plsc.load_gather(rows_v, [rows + 1, col1])
            z1 = plsc.load_gather(rows_v, [rows + 1, col2])
            dx = x0 - x1
            dy = y0 - y1
            dz = z0 - z1
            r2 = dx * dx + dy * dy + dz * dz
            r = jnp.sqrt(r2)
            b0g = b0_v[pl.ds(g * L, L)]
            kg = k_v[pl.ds(g * L, L)]
            d = r - b0g
            return acc_in + d * d * kg * 0.5

        return lax.fori_loop(0, GROUPS, group_step, acc)

    acc = lax.fori_loop(0, N_CHUNKS, chunk_step, jnp.zeros((L,), _f32))

    acc_v[...] = acc
    pltpu.sync_copy(acc_v, partials_sh.at[sid])
    plsc.subcore_barrier()

    @pl.when(sid == 0)
    def _():
        pltpu.sync_copy(partials_sh, red_v)
        tot = jnp.zeros((L,), _f32)
        for s in range(NS):
            tot = tot + red_v[s, :]
        tot_v[...] = tot
        pltpu.sync_copy(tot_v, out_hbm.at[cid])


@functools.partial(
    pl.kernel,
    out_type=jax.ShapeDtypeStruct((NC, L), _f32),
    mesh=plsc.VectorSubcoreMesh(
        core_axis_name="c", subcore_axis_name="s",
        num_cores=NC, num_subcores=NS),
    scratch_types=dict(
        coords_sh=pltpu.VMEM_SHARED((N_ATOMS, 4), _f32),
        partials_sh=pltpu.VMEM_SHARED((NS, L), _f32),
        bidx_v=pltpu.VMEM((2 * CHUNK,), _i32),
        b0_v=pltpu.VMEM((CHUNK,), _f32),
        k_v=pltpu.VMEM((CHUNK,), _f32),
        rows_v=pltpu.VMEM((2 * CHUNK, 4), _f32),
        red_v=pltpu.VMEM((NS, L), _f32),
        acc_v=pltpu.VMEM((L,), _f32),
        tot_v=pltpu.VMEM((L,), _f32),
        sem0=pltpu.SemaphoreType.DMA,
    ),
)
def _harmonic_bond_sc(coords_hbm, bidx_hbm, b0_hbm, k_hbm, out_hbm, **scr):
    _sc_body(coords_hbm, bidx_hbm, b0_hbm, k_hbm, out_hbm,
             scr["coords_sh"], scr["partials_sh"], scr["bidx_v"],
             scr["b0_v"], scr["k_v"], scr["rows_v"], scr["red_v"],
             scr["acc_v"], scr["tot_v"], scr["sem0"])


def kernel(coords, bonds, b0, k):
    coords4 = jnp.concatenate(
        [coords.astype(_f32),
         jnp.zeros((coords.shape[0], 1), _f32)], axis=1)
    bidx = bonds.astype(_i32).reshape(-1)
    partials = _harmonic_bond_sc(coords4, bidx, b0, k)
    return jnp.sum(partials)


# speedup vs baseline: 30.2883x; 30.2883x over previous
import functools

import jax
import jax.numpy as jnp
from jax import lax
from jax.experimental import pallas as pl
from jax.experimental.pallas import tpu as pltpu
from jax.experimental.pallas import tpu_sc as plsc

NC = 2
NS = 16
L = 16
NW = NC * NS

N_ATOMS = 100000
N_BONDS = 1600000
BT = N_BONDS // NW
CHUNK = 2000
N_CHUNKS = BT // CHUNK
GROUPS = CHUNK // L

_i32 = jnp.int32
_f32 = jnp.float32


def _sc_body(xs_hbm, ys_hbm, zs_hbm, i0_hbm, i1_hbm, b0_hbm, k_hbm, out_hbm,
             xs_sh, ys_sh, zs_sh, partials_sh,
             i0_v, i1_v, b0_v, k_v, x0_v, y0_v, z0_v, x1_v, y1_v, z1_v,
             red_v, acc_v, sem0):
    cid = lax.axis_index("c")
    sid = lax.axis_index("s")
    wid = cid * NS + sid

    @pl.when(sid == 0)
    def _():
        pltpu.sync_copy(xs_hbm, xs_sh)
        pltpu.sync_copy(ys_hbm, ys_sh)
        pltpu.sync_copy(zs_hbm, zs_sh)

    plsc.subcore_barrier()

    def chunk_step(t, acc):
        base = wid * BT + t * CHUNK
        pltpu.sync_copy(i0_hbm.at[pl.ds(base, CHUNK)], i0_v)
        pltpu.sync_copy(i1_hbm.at[pl.ds(base, CHUNK)], i1_v)
        pltpu.sync_copy(b0_hbm.at[pl.ds(base, CHUNK)], b0_v)
        pltpu.sync_copy(k_hbm.at[pl.ds(base, CHUNK)], k_v)
        copies = [
            pltpu.async_copy(xs_sh.at[i0_v], x0_v, sem0),
            pltpu.async_copy(ys_sh.at[i0_v], y0_v, sem0),
            pltpu.async_copy(zs_sh.at[i0_v], z0_v, sem0),
            pltpu.async_copy(xs_sh.at[i1_v], x1_v, sem0),
            pltpu.async_copy(ys_sh.at[i1_v], y1_v, sem0),
            pltpu.async_copy(zs_sh.at[i1_v], z1_v, sem0),
        ]
        for c in copies:
            c.wait()

        def group_step(g, acc_in):
            sl = pl.ds(g * L, L)
            dx = x0_v[sl] - x1_v[sl]
            dy = y0_v[sl] - y1_v[sl]
            dz = z0_v[sl] - z1_v[sl]
            r2 = dx * dx + dy * dy + dz * dz
            r2c = jnp.maximum(r2, 1e-30)
            seed = jnp.full((L,), 0x5F3759DF, _i32) - lax.shift_right_logical(
                plsc.bitcast(r2c, _i32), 1)
            y = plsc.bitcast(seed, _f32)
            h = 0.5 * r2c
            y = y * (1.5 - h * y * y)
            y = y * (1.5 - h * y * y)
            y = y * (1.5 - h * y * y)
            r = r2 * y
            d = r - b0_v[sl]
            return acc_in + d * d * k_v[sl] * 0.5

        return lax.fori_loop(0, GROUPS, group_step, acc)

    acc = lax.fori_loop(0, N_CHUNKS, chunk_step, jnp.zeros((L,), _f32))

    acc_v[...] = acc
    pltpu.sync_copy(acc_v, partials_sh.at[sid])
    plsc.subcore_barrier()

    @pl.when(sid == 0)
    def _():
        pltpu.sync_copy(partials_sh, red_v)
        tot = jnp.zeros((L,), _f32)
        for s in range(NS):
            tot = tot + red_v[s, :]
        acc_v[...] = tot
        pltpu.sync_copy(acc_v, out_hbm.at[cid])


@functools.partial(
    pl.kernel,
    out_type=jax.ShapeDtypeStruct((NC, L), _f32),
    mesh=plsc.VectorSubcoreMesh(
        core_axis_name="c", subcore_axis_name="s",
        num_cores=NC, num_subcores=NS),
    scratch_types=dict(
        xs_sh=pltpu.VMEM_SHARED((N_ATOMS,), _f32),
        ys_sh=pltpu.VMEM_SHARED((N_ATOMS,), _f32),
        zs_sh=pltpu.VMEM_SHARED((N_ATOMS,), _f32),
        partials_sh=pltpu.VMEM_SHARED((NS, L), _f32),
        i0_v=pltpu.VMEM((CHUNK,), _i32),
        i1_v=pltpu.VMEM((CHUNK,), _i32),
        b0_v=pltpu.VMEM((CHUNK,), _f32),
        k_v=pltpu.VMEM((CHUNK,), _f32),
        x0_v=pltpu.VMEM((CHUNK,), _f32),
        y0_v=pltpu.VMEM((CHUNK,), _f32),
        z0_v=pltpu.VMEM((CHUNK,), _f32),
        x1_v=pltpu.VMEM((CHUNK,), _f32),
        y1_v=pltpu.VMEM((CHUNK,), _f32),
        z1_v=pltpu.VMEM((CHUNK,), _f32),
        red_v=pltpu.VMEM((NS, L), _f32),
        acc_v=pltpu.VMEM((L,), _f32),
        sem0=pltpu.SemaphoreType.DMA,
    ),
    compiler_params=pltpu.CompilerParams(
        needs_layout_passes=False, use_tc_tiling_on_sc=False),
)
def _harmonic_bond_sc(xs_hbm, ys_hbm, zs_hbm, i0_hbm, i1_hbm, b0_hbm, k_hbm,
                      out_hbm, **scr):
    _sc_body(xs_hbm, ys_hbm, zs_hbm, i0_hbm, i1_hbm, b0_hbm, k_hbm, out_hbm,
             scr["xs_sh"], scr["ys_sh"], scr["zs_sh"], scr["partials_sh"],
             scr["i0_v"], scr["i1_v"], scr["b0_v"], scr["k_v"],
             scr["x0_v"], scr["y0_v"], scr["z0_v"],
             scr["x1_v"], scr["y1_v"], scr["z1_v"],
             scr["red_v"], scr["acc_v"], scr["sem0"])


def kernel(coords, bonds, b0, k):
    coords = coords.astype(_f32)
    bonds = bonds.astype(_i32)
    partials = _harmonic_bond_sc(
        coords[:, 0], coords[:, 1], coords[:, 2],
        bonds[:, 0], bonds[:, 1], b0, k)
    return jnp.sum(partials)


# speedup vs baseline: 42.1725x; 1.3924x over previous
import functools

import jax
import jax.numpy as jnp
from jax import lax
from jax.experimental import pallas as pl
from jax.experimental.pallas import tpu as pltpu
from jax.experimental.pallas import tpu_sc as plsc

NC = 2
NS = 16
L = 16
NW = NC * NS

N_ATOMS = 100000
N_BONDS = 1600000
BT = N_BONDS // NW
CHUNK = 2000
N_CHUNKS = BT // CHUNK
GROUPS = CHUNK // L

_i32 = jnp.int32
_f32 = jnp.float32


def _sc_body(xs_hbm, ys_hbm, zs_hbm, i0_hbm, i1_hbm, b0_hbm, k_hbm, out_hbm,
             xs_sh, ys_sh, zs_sh, partials_sh, bufs, red_v, acc_v, sems):
    cid = lax.axis_index("c")
    sid = lax.axis_index("s")
    wid = cid * NS + sid

    @pl.when(sid == 0)
    def _():
        pltpu.sync_copy(xs_hbm, xs_sh)
        pltpu.sync_copy(ys_hbm, ys_sh)
        pltpu.sync_copy(zs_hbm, zs_sh)

    plsc.subcore_barrier()

    def issue_lin(t):
        p = t % 2
        i0_v, i1_v, b0_v, k_v = bufs[p][:4]
        base = wid * BT + t * CHUNK
        sem = sems[p]
        return [
            pltpu.async_copy(i0_hbm.at[pl.ds(base, CHUNK)], i0_v, sem),
            pltpu.async_copy(i1_hbm.at[pl.ds(base, CHUNK)], i1_v, sem),
            pltpu.async_copy(b0_hbm.at[pl.ds(base, CHUNK)], b0_v, sem),
            pltpu.async_copy(k_hbm.at[pl.ds(base, CHUNK)], k_v, sem),
        ]

    def issue_gather(t):
        p = t % 2
        i0_v, i1_v = bufs[p][0], bufs[p][1]
        x0_v, y0_v, z0_v, x1_v, y1_v, z1_v = bufs[p][4:]
        sem = sems[2 + p]
        return [
            pltpu.async_copy(xs_sh.at[i0_v], x0_v, sem),
            pltpu.async_copy(ys_sh.at[i0_v], y0_v, sem),
            pltpu.async_copy(zs_sh.at[i0_v], z0_v, sem),
            pltpu.async_copy(xs_sh.at[i1_v], x1_v, sem),
            pltpu.async_copy(ys_sh.at[i1_v], y1_v, sem),
            pltpu.async_copy(zs_sh.at[i1_v], z1_v, sem),
        ]

    def compute(t, acc):
        p = t % 2
        b0_v, k_v = bufs[p][2], bufs[p][3]
        x0_v, y0_v, z0_v, x1_v, y1_v, z1_v = bufs[p][4:]

        def group_step(g, acc_in):
            sl = pl.ds(g * L, L)
            dx = x0_v[sl] - x1_v[sl]
            dy = y0_v[sl] - y1_v[sl]
            dz = z0_v[sl] - z1_v[sl]
            r2 = dx * dx + dy * dy + dz * dz
            r2c = jnp.maximum(r2, 1e-30)
            seed = jnp.full((L,), 0x5F3759DF, _i32) - lax.shift_right_logical(
                plsc.bitcast(r2c, _i32), 1)
            y = plsc.bitcast(seed, _f32)
            h = 0.5 * r2c
            y = y * (1.5 - h * y * y)
            y = y * (1.5 - h * y * y)
            y = y * (1.5 - h * y * y)
            r = r2 * y
            d = r - b0_v[sl]
            return acc_in + d * d * k_v[sl] * 0.5

        return lax.fori_loop(0, GROUPS, group_step, acc, unroll=5)

    acc = jnp.zeros((L,), _f32)
    lin = {0: issue_lin(0)}
    for c in lin[0]:
        c.wait()
    gat = {0: issue_gather(0)}
    lin[1] = issue_lin(1)
    for t in range(N_CHUNKS):
        if t + 1 < N_CHUNKS:
            for c in lin.pop(t + 1):
                c.wait()
            gat[t + 1] = issue_gather(t + 1)
        for c in gat.pop(t):
            c.wait()
        acc = compute(t, acc)
        if t + 2 < N_CHUNKS:
            lin[t + 2] = issue_lin(t + 2)

    acc_v[...] = acc
    pltpu.sync_copy(acc_v, partials_sh.at[sid])
    plsc.subcore_barrier()

    @pl.when(sid == 0)
    def _():
        pltpu.sync_copy(partials_sh, red_v)
        tot = jnp.zeros((L,), _f32)
        for s in range(NS):
            tot = tot + red_v[s, :]
        acc_v[...] = tot
        pltpu.sync_copy(acc_v, out_hbm.at[cid])


def _buf(dtype):
    return pltpu.VMEM((CHUNK,), dtype)


@functools.partial(
    pl.kernel,
    out_type=jax.ShapeDtypeStruct((NC, L), _f32),
    mesh=plsc.VectorSubcoreMesh(
        core_axis_name="c", subcore_axis_name="s",
        num_cores=NC, num_subcores=NS),
    scratch_types=dict(
        xs_sh=pltpu.VMEM_SHARED((N_ATOMS,), _f32),
        ys_sh=pltpu.VMEM_SHARED((N_ATOMS,), _f32),
        zs_sh=pltpu.VMEM_SHARED((N_ATOMS,), _f32),
        partials_sh=pltpu.VMEM_SHARED((NS, L), _f32),
        bufs=[[_buf(_i32), _buf(_i32), _buf(_f32), _buf(_f32),
               _buf(_f32), _buf(_f32), _buf(_f32),
               _buf(_f32), _buf(_f32), _buf(_f32)] for _ in range(2)],
        red_v=pltpu.VMEM((NS, L), _f32),
        acc_v=pltpu.VMEM((L,), _f32),
        sems=[pltpu.SemaphoreType.DMA for _ in range(4)],
    ),
    compiler_params=pltpu.CompilerParams(
        needs_layout_passes=False, use_tc_tiling_on_sc=False),
)
def _harmonic_bond_sc(xs_hbm, ys_hbm, zs_hbm, i0_hbm, i1_hbm, b0_hbm, k_hbm,
                      out_hbm, **scr):
    _sc_body(xs_hbm, ys_hbm, zs_hbm, i0_hbm, i1_hbm, b0_hbm, k_hbm, out_hbm,
             scr["xs_sh"], scr["ys_sh"], scr["zs_sh"], scr["partials_sh"],
             scr["bufs"], scr["red_v"], scr["acc_v"], scr["sems"])


def kernel(coords, bonds, b0, k):
    coords = coords.astype(_f32)
    bonds = bonds.astype(_i32)
    partials = _harmonic_bond_sc(
        coords[:, 0], coords[:, 1], coords[:, 2],
        bonds[:, 0], bonds[:, 1], b0, k)
    return jnp.sum(partials)
